# baseline (speedup 1.0000x reference)
import jax
import jax.numpy as jnp
from jax.experimental import pallas as pl
from jax.experimental.pallas import tpu as pltpu

_BM = 400


def _gcn_kernel(adj_ref, x_ref, w1_ref, b1_ref, w2_ref, b2_ref,
                out_ref, xw_ref, g_ref):
    p = pl.program_id(0)
    i = pl.program_id(1)

    @pl.when(jnp.logical_and(p == 0, i == 0))
    def _():
        xw = jax.lax.dot_general(
            x_ref[...], w1_ref[...].astype(jnp.bfloat16),
            (((1,), (1,)), ((), ())), preferred_element_type=jnp.float32)
        xw_ref[...] = xw.astype(jnp.bfloat16)

    a = adj_ref[0].astype(jnp.bfloat16)

    @pl.when(p == 0)
    def _():
        h = jnp.dot(a, xw_ref[...], preferred_element_type=jnp.float32)
        h = jnp.maximum(h + b1_ref[...], 0.0)
        g = jax.lax.dot_general(
            h.astype(jnp.bfloat16), w2_ref[...].astype(jnp.bfloat16),
            (((1,), (1,)), ((), ())), preferred_element_type=jnp.float32)
        g_ref[pl.ds(i * _BM, _BM), :] = g.astype(jnp.bfloat16)

    @pl.when(p == 1)
    def _():
        h = jnp.dot(a, g_ref[...], preferred_element_type=jnp.float32)
        out_ref[0] = jnp.maximum(h + b2_ref[...], 0.0)


def kernel(x, adj, W1, b1, W2, b2):
    n, d = x.shape
    h_dim = W1.shape[0]
    o_dim = W2.shape[0]
    nb = n // _BM
    adj3 = adj.reshape(nb, _BM, n)
    xb = x.astype(jnp.bfloat16)

    out = pl.pallas_call(
        _gcn_kernel,
        grid=(2, nb),
        in_specs=[
            pl.BlockSpec((1, _BM, n), lambda p, i: (i, 0, 0)),
            pl.BlockSpec((n, d), lambda p, i: (0, 0)),
            pl.BlockSpec((h_dim, d), lambda p, i: (0, 0)),
            pl.BlockSpec((1, h_dim), lambda p, i: (0, 0)),
            pl.BlockSpec((o_dim, h_dim), lambda p, i: (0, 0)),
            pl.BlockSpec((1, o_dim), lambda p, i: (0, 0)),
        ],
        out_specs=pl.BlockSpec((1, _BM, o_dim), lambda p, i: (i * p, 0, 0)),
        out_shape=jax.ShapeDtypeStruct((nb, _BM, o_dim), jnp.float32),
        scratch_shapes=[
            pltpu.VMEM((n, h_dim), jnp.bfloat16),
            pltpu.VMEM((n, o_dim), jnp.bfloat16),
        ],
        compiler_params=pltpu.CompilerParams(
            vmem_limit_bytes=64 * 1024 * 1024),
    )(adj3, xb, W1, b1.reshape(1, h_dim), W2, b2.reshape(1, o_dim))

    return out.reshape(n, o_dim)


# speedup vs baseline: 1.0125x; 1.0125x over previous
import jax
import jax.numpy as jnp
from jax.experimental import pallas as pl
from jax.experimental.pallas import tpu as pltpu

_BM = 400


def _gcn_kernel(adj_ref, x_ref, w1_ref, b1_ref, w2_ref, b2_ref,
                out_ref, xw_ref, g_ref):
    p = pl.program_id(0)
    i = pl.program_id(1)

    @pl.when(jnp.logical_and(p == 0, i == 0))
    def _():
        xw = jax.lax.dot_general(
            x_ref[...].astype(jnp.bfloat16), w1_ref[...].astype(jnp.bfloat16),
            (((1,), (1,)), ((), ())), preferred_element_type=jnp.float32)
        xw_ref[...] = xw.astype(jnp.bfloat16)

    a = adj_ref[0].astype(jnp.bfloat16)

    @pl.when(p == 0)
    def _():
        h = jnp.dot(a, xw_ref[...], preferred_element_type=jnp.float32)
        h = jnp.maximum(h + b1_ref[...], 0.0)
        g = jax.lax.dot_general(
            h.astype(jnp.bfloat16), w2_ref[...].astype(jnp.bfloat16),
            (((1,), (1,)), ((), ())), preferred_element_type=jnp.float32)
        g_ref[pl.ds(i * _BM, _BM), :] = g.astype(jnp.bfloat16)

    @pl.when(p == 1)
    def _():
        h = jnp.dot(a, g_ref[...], preferred_element_type=jnp.float32)
        out_ref[0] = jnp.maximum(h + b2_ref[...], 0.0)


def kernel(x, adj, W1, b1, W2, b2):
    n, d = x.shape
    h_dim = W1.shape[0]
    o_dim = W2.shape[0]
    nb = n // _BM
    adj3 = adj.reshape(nb, _BM, n)

    out = pl.pallas_call(
        _gcn_kernel,
        grid=(2, nb),
        in_specs=[
            pl.BlockSpec((1, _BM, n), lambda p, i: (i, 0, 0)),
            pl.BlockSpec((n, d), lambda p, i: (0, 0)),
            pl.BlockSpec((h_dim, d), lambda p, i: (0, 0)),
            pl.BlockSpec((1, h_dim), lambda p, i: (0, 0)),
            pl.BlockSpec((o_dim, h_dim), lambda p, i: (0, 0)),
            pl.BlockSpec((1, o_dim), lambda p, i: (0, 0)),
        ],
        out_specs=pl.BlockSpec((1, _BM, o_dim), lambda p, i: (i * p, 0, 0)),
        out_shape=jax.ShapeDtypeStruct((nb, _BM, o_dim), jnp.float32),
        scratch_shapes=[
            pltpu.VMEM((n, h_dim), jnp.bfloat16),
            pltpu.VMEM((n, o_dim), jnp.bfloat16),
        ],
        compiler_params=pltpu.CompilerParams(
            vmem_limit_bytes=64 * 1024 * 1024),
    )(adj3, x, W1, b1.reshape(1, h_dim), W2, b2.reshape(1, o_dim))

    return out.reshape(n, o_dim)


# speedup vs baseline: 1.0196x; 1.0071x over previous
import jax
import jax.numpy as jnp
from jax.experimental import pallas as pl
from jax.experimental.pallas import tpu as pltpu

_BM = 400


def _gcn_kernel(adj_ref, x_ref, w1_ref, b1_ref, w2_ref, b2_ref,
                out_ref, xw_ref, h1_ref, g_ref):
    p = pl.program_id(0)
    i = pl.program_id(1)

    @pl.when(jnp.logical_and(p == 0, i == 0))
    def _():
        xw = jax.lax.dot_general(
            x_ref[...].astype(jnp.bfloat16), w1_ref[...].astype(jnp.bfloat16),
            (((1,), (1,)), ((), ())), preferred_element_type=jnp.float32)
        xw_ref[...] = xw.astype(jnp.bfloat16)

    a = adj_ref[0].astype(jnp.bfloat16)

    @pl.when(p == 0)
    def _():
        h = jnp.dot(a, xw_ref[...], preferred_element_type=jnp.float32)
        h = jnp.maximum(h + b1_ref[...], 0.0)
        h1_ref[pl.ds(i * _BM, _BM), :] = h.astype(jnp.bfloat16)

    @pl.when(jnp.logical_and(p == 1, i == 0))
    def _():
        g = jax.lax.dot_general(
            h1_ref[...], w2_ref[...].astype(jnp.bfloat16),
            (((1,), (1,)), ((), ())), preferred_element_type=jnp.float32)
        g_ref[...] = g.astype(jnp.bfloat16)

    @pl.when(p == 1)
    def _():
        h = jnp.dot(a, g_ref[...], preferred_element_type=jnp.float32)
        out_ref[0] = jnp.maximum(h + b2_ref[...], 0.0)


def kernel(x, adj, W1, b1, W2, b2):
    n, d = x.shape
    h_dim = W1.shape[0]
    o_dim = W2.shape[0]
    nb = n // _BM
    adj3 = adj.reshape(nb, _BM, n)

    out = pl.pallas_call(
        _gcn_kernel,
        grid=(2, nb),
        in_specs=[
            pl.BlockSpec((1, _BM, n), lambda p, i: (i, 0, 0)),
            pl.BlockSpec((n, d), lambda p, i: (0, 0)),
            pl.BlockSpec((h_dim, d), lambda p, i: (0, 0)),
            pl.BlockSpec((1, h_dim), lambda p, i: (0, 0)),
            pl.BlockSpec((o_dim, h_dim), lambda p, i: (0, 0)),
            pl.BlockSpec((1, o_dim), lambda p, i: (0, 0)),
        ],
        out_specs=pl.BlockSpec((1, _BM, o_dim), lambda p, i: (i * p, 0, 0)),
        out_shape=jax.ShapeDtypeStruct((nb, _BM, o_dim), jnp.float32),
        scratch_shapes=[
            pltpu.VMEM((n, h_dim), jnp.bfloat16),
            pltpu.VMEM((n, h_dim), jnp.bfloat16),
            pltpu.VMEM((n, o_dim), jnp.bfloat16),
        ],
        compiler_params=pltpu.CompilerParams(
            vmem_limit_bytes=64 * 1024 * 1024),
    )(adj3, x, W1, b1.reshape(1, h_dim), W2, b2.reshape(1, o_dim))

    return out.reshape(n, o_dim)
